# baseline (speedup 1.0000x reference)
import functools

import jax
import jax.numpy as jnp
from jax import lax
from jax.experimental import pallas as pl
from jax.experimental.pallas import tpu as pltpu
from jax.experimental.pallas import tpu_sc as plsc

B, S, D, N_DEC = 1024, 50, 128, 8
SD = S * D
NC, NS = 2, 16
NW = NC * NS
RPW = B // NW


def _select_body(o0, o1, o2, o3, o4, o5, o6, o7, cid_hbm, out_hbm, cid_v, sem):
    srcs = [o0, o1, o2, o3, o4, o5, o6, o7]
    wid = lax.axis_index("s") * NC + lax.axis_index("c")
    base = wid * RPW
    pltpu.sync_copy(cid_hbm.at[pl.ds(base, RPW)], cid_v.at[pl.ds(0, RPW)])

    def issue(j, carry):
        c = cid_v[pl.ds(j, 16)][0]
        row = base + j
        for i in range(N_DEC):
            @pl.when(c == i)
            def _():
                pltpu.make_async_copy(
                    srcs[i].at[row], out_hbm.at[row], sem).start()
        return carry

    lax.fori_loop(0, RPW, issue, 0)

    def drain(j, carry):
        pltpu.make_async_copy(o0.at[0], out_hbm.at[0], sem).wait()
        return carry

    lax.fori_loop(0, RPW, drain, 0)


@jax.jit
def _select(outs_flat, cid):
    mesh = plsc.VectorSubcoreMesh(core_axis_name="c", subcore_axis_name="s")
    return pl.kernel(
        _select_body,
        out_type=jax.ShapeDtypeStruct((B, SD), jnp.float32),
        mesh=mesh,
        scratch_types=[
            pltpu.VMEM((RPW + 16,), jnp.int32),
            pltpu.SemaphoreType.DMA,
        ],
    )(*outs_flat, cid)


def kernel(out0, out1, out2, out3, out4, out5, out6, out7, comp_id):
    outs = [o.reshape(B, SD) for o in
            (out0, out1, out2, out3, out4, out5, out6, out7)]
    cid = comp_id.reshape(B)
    res = _select(outs, cid)
    return res.reshape(B, S, D)


# speedup vs baseline: 2.9895x; 2.9895x over previous
import jax
import jax.numpy as jnp
from jax import lax
from jax.experimental import pallas as pl
from jax.experimental.pallas import tpu as pltpu
from jax.experimental.pallas import tpu_sc as plsc

B, S, D, N_DEC = 1024, 50, 128, 8
SD = S * D
NC, NS = 2, 16
NW = NC * NS
RPW = B // NW
BURST = 16


def _select_body(o0, o1, o2, o3, o4, o5, o6, o7, cid_hbm, out_hbm,
                 cid_v, buf, gsem, ssem):
    srcs = [o0, o1, o2, o3, o4, o5, o6, o7]
    wid = lax.axis_index("s") * NC + lax.axis_index("c")
    base = wid * RPW
    pltpu.sync_copy(cid_hbm.at[pl.ds(base, RPW)], cid_v.at[pl.ds(0, RPW)])

    for h in range(RPW // BURST):
        hbase = base + h * BURST

        def gissue(b, carry):
            c = cid_v[pl.ds(h * BURST + b, 16)][0]
            row = hbase + b
            for i in range(N_DEC):
                @pl.when(c == i)
                def _():
                    pltpu.make_async_copy(
                        srcs[i].at[row], buf.at[b], gsem).start()
            return carry

        lax.fori_loop(0, BURST, gissue, 0)

        def gdrain(b, carry):
            pltpu.make_async_copy(o0.at[0], buf.at[0], gsem).wait()
            return carry

        lax.fori_loop(0, BURST, gdrain, 0)

        def sissue(b, carry):
            pltpu.make_async_copy(buf.at[b], out_hbm.at[hbase + b],
                                  ssem).start()
            return carry

        lax.fori_loop(0, BURST, sissue, 0)

        def sdrain(b, carry):
            pltpu.make_async_copy(o0.at[0], out_hbm.at[0], ssem).wait()
            return carry

        lax.fori_loop(0, BURST, sdrain, 0)


@jax.jit
def _select(outs_flat, cid):
    mesh = plsc.VectorSubcoreMesh(core_axis_name="c", subcore_axis_name="s")
    return pl.kernel(
        _select_body,
        out_type=jax.ShapeDtypeStruct((B, SD), jnp.float32),
        mesh=mesh,
        scratch_types=[
            pltpu.VMEM((RPW + 16,), jnp.int32),
            pltpu.VMEM((BURST, SD), jnp.float32),
            pltpu.SemaphoreType.DMA,
            pltpu.SemaphoreType.DMA,
        ],
    )(*outs_flat, cid)


def kernel(out0, out1, out2, out3, out4, out5, out6, out7, comp_id):
    outs = [o.reshape(B, SD) for o in
            (out0, out1, out2, out3, out4, out5, out6, out7)]
    cid = comp_id.reshape(B)
    res = _select(outs, cid)
    return res.reshape(B, S, D)


# speedup vs baseline: 4.1022x; 1.3722x over previous
import jax
import jax.numpy as jnp
from jax import lax
from jax.experimental import pallas as pl
from jax.experimental.pallas import tpu as pltpu
from jax.experimental.pallas import tpu_sc as plsc

B, S, D, N_DEC = 1024, 50, 128, 8
NC, NS = 2, 16
NW = NC * NS
RPW = B // NW
BURST = 16


def _select_body(o0, o1, o2, o3, o4, o5, o6, o7, cid_hbm, out_hbm,
                 cid_v, buf, gsem, ssem):
    srcs = [o0, o1, o2, o3, o4, o5, o6, o7]
    wid = lax.axis_index("s") * NC + lax.axis_index("c")
    base = wid * RPW
    pltpu.sync_copy(cid_hbm.at[pl.ds(base, RPW)], cid_v.at[pl.ds(0, RPW)])

    for h in range(RPW // BURST):
        hbase = base + h * BURST

        def gissue(b, carry):
            c = cid_v[pl.ds(h * BURST + b, 16)][0]
            row = hbase + b
            for i in range(N_DEC):
                @pl.when(c == i)
                def _():
                    pltpu.make_async_copy(
                        srcs[i].at[row], buf.at[b], gsem).start()
            return carry

        lax.fori_loop(0, BURST, gissue, 0)

        def gdrain(b, carry):
            pltpu.make_async_copy(o0.at[0], buf.at[0], gsem).wait()
            return carry

        lax.fori_loop(0, BURST, gdrain, 0)

        def sissue(b, carry):
            pltpu.make_async_copy(buf.at[b], out_hbm.at[hbase + b],
                                  ssem).start()
            return carry

        lax.fori_loop(0, BURST, sissue, 0)

        def sdrain(b, carry):
            pltpu.make_async_copy(o0.at[0], out_hbm.at[0], ssem).wait()
            return carry

        lax.fori_loop(0, BURST, sdrain, 0)


@jax.jit
def _select(outs, cid):
    mesh = plsc.VectorSubcoreMesh(core_axis_name="c", subcore_axis_name="s")
    return pl.kernel(
        _select_body,
        out_type=jax.ShapeDtypeStruct((B, S, D), jnp.float32),
        mesh=mesh,
        scratch_types=[
            pltpu.VMEM((RPW + 16,), jnp.int32),
            pltpu.VMEM((BURST, S, D), jnp.float32),
            pltpu.SemaphoreType.DMA,
            pltpu.SemaphoreType.DMA,
        ],
        compiler_params=pltpu.CompilerParams(use_tc_tiling_on_sc=True),
    )(*outs, cid)


def kernel(out0, out1, out2, out3, out4, out5, out6, out7, comp_id):
    outs = [out0, out1, out2, out3, out4, out5, out6, out7]
    cid = comp_id.reshape(B)
    return _select(outs, cid)


# speedup vs baseline: 28.3691x; 6.9156x over previous
import jax
import jax.numpy as jnp
from jax import lax
from jax.experimental import pallas as pl
from jax.experimental.pallas import tpu as pltpu
from jax.experimental.pallas import tpu_sc as plsc

B, S, D, N_DEC = 1024, 50, 128, 8
NC, NS = 2, 16
NW = NC * NS
RPW = B // NW
NBUF = 8


def _select_body(o0, o1, o2, o3, o4, o5, o6, o7, cid_hbm, out_hbm,
                 cid_v, buf, gsem, ssem):
    srcs = [o0, o1, o2, o3, o4, o5, o6, o7]
    wid = lax.axis_index("s") * NC + lax.axis_index("c")
    base = wid * RPW
    pltpu.sync_copy(cid_hbm.at[pl.ds(base, RPW)], cid_v.at[pl.ds(0, RPW)])

    def gather(j, slot):
        c = cid_v[pl.ds(j, 16)][0]
        for i in range(N_DEC):
            @pl.when(c == i)
            def _():
                pltpu.make_async_copy(
                    srcs[i].at[:, base + j], buf.at[:, slot],
                    gsem.at[slot]).start()

    for k in range(NBUF):
        gather(k, k)

    def step(j, carry):
        slot = lax.rem(j, NBUF)
        pltpu.make_async_copy(o0.at[:, 0], buf.at[:, 0],
                              gsem.at[slot]).wait()
        pltpu.make_async_copy(buf.at[:, slot], out_hbm.at[:, base + j],
                              ssem.at[slot]).start()

        @pl.when(j + NBUF < RPW)
        def _():
            pltpu.make_async_copy(o0.at[:, 0], out_hbm.at[:, 0],
                                  ssem.at[slot]).wait()
            gather(j + NBUF, slot)

        return carry

    lax.fori_loop(0, RPW, step, 0)

    for k in range(NBUF):
        slot = (RPW - NBUF + k) % NBUF
        pltpu.make_async_copy(o0.at[:, 0], out_hbm.at[:, 0],
                              ssem.at[slot]).wait()


@jax.jit
def _select(outs, cid):
    mesh = plsc.VectorSubcoreMesh(core_axis_name="c", subcore_axis_name="s")
    return pl.kernel(
        _select_body,
        out_type=jax.ShapeDtypeStruct((S, B, D), jnp.float32),
        mesh=mesh,
        scratch_types=[
            pltpu.VMEM((RPW + 32,), jnp.int32),
            pltpu.VMEM((S, NBUF, D), jnp.float32),
            pltpu.SemaphoreType.DMA((NBUF,)),
            pltpu.SemaphoreType.DMA((NBUF,)),
        ],
        compiler_params=pltpu.CompilerParams(use_tc_tiling_on_sc=True),
    )(*outs, cid)


def kernel(out0, out1, out2, out3, out4, out5, out6, out7, comp_id):
    outs = [jnp.transpose(o, (1, 0, 2)) for o in
            (out0, out1, out2, out3, out4, out5, out6, out7)]
    cid = comp_id.reshape(B)
    res = _select(outs, cid)
    return jnp.transpose(res, (1, 0, 2))


# speedup vs baseline: 29.1785x; 1.0285x over previous
import jax
import jax.numpy as jnp
from jax import lax
from jax.experimental import pallas as pl
from jax.experimental.pallas import tpu as pltpu
from jax.experimental.pallas import tpu_sc as plsc

B, S, D, N_DEC = 1024, 50, 128, 8
NC, NS = 2, 16
NW = NC * NS
RPW = B // NW
NBUF = 8
LOOK = 4


def _select_body(o0, o1, o2, o3, o4, o5, o6, o7, cid_hbm, out_hbm,
                 cid_v, buf, gsem, ssem):
    srcs = [o0, o1, o2, o3, o4, o5, o6, o7]
    wid = lax.axis_index("s") * NC + lax.axis_index("c")
    base = wid * RPW
    pltpu.sync_copy(cid_hbm.at[pl.ds(base, RPW)], cid_v.at[pl.ds(0, RPW)])

    def gather(j, slot):
        c = cid_v[pl.ds(j, 16)][0]
        for i in range(N_DEC):
            @pl.when(c == i)
            def _():
                pltpu.make_async_copy(
                    srcs[i].at[:, base + j], buf.at[:, slot],
                    gsem.at[slot]).start()

    for k in range(LOOK):
        gather(k, k)

    def step(j, carry):
        slot = lax.rem(j, NBUF)
        nslot = lax.rem(j + LOOK, NBUF)

        @pl.when(jnp.logical_and(j >= LOOK, j + LOOK < RPW))
        def _():
            pltpu.make_async_copy(o0.at[:, 0], out_hbm.at[:, 0],
                                  ssem.at[nslot]).wait()

        pltpu.make_async_copy(o0.at[:, 0], buf.at[:, 0],
                              gsem.at[slot]).wait()
        pltpu.make_async_copy(buf.at[:, slot], out_hbm.at[:, base + j],
                              ssem.at[slot]).start()

        @pl.when(j + LOOK < RPW)
        def _():
            gather(j + LOOK, nslot)

        return carry

    lax.fori_loop(0, RPW, step, 0)

    for k in range(NBUF):
        pltpu.make_async_copy(o0.at[:, 0], out_hbm.at[:, 0],
                              ssem.at[k]).wait()


@jax.jit
def _select(outs, cid):
    mesh = plsc.VectorSubcoreMesh(core_axis_name="c", subcore_axis_name="s")
    return pl.kernel(
        _select_body,
        out_type=jax.ShapeDtypeStruct((S, B, D), jnp.float32),
        mesh=mesh,
        scratch_types=[
            pltpu.VMEM((RPW + 32,), jnp.int32),
            pltpu.VMEM((S, NBUF, D), jnp.float32),
            pltpu.SemaphoreType.DMA((NBUF,)),
            pltpu.SemaphoreType.DMA((NBUF,)),
        ],
        compiler_params=pltpu.CompilerParams(use_tc_tiling_on_sc=True),
    )(*outs, cid)


def kernel(out0, out1, out2, out3, out4, out5, out6, out7, comp_id):
    outs = [jnp.transpose(o, (1, 0, 2)) for o in
            (out0, out1, out2, out3, out4, out5, out6, out7)]
    cid = comp_id.reshape(B)
    res = _select(outs, cid)
    return jnp.transpose(res, (1, 0, 2))
